# Initial kernel scaffold; baseline (speedup 1.0000x reference)
import jax
import jax.numpy as jnp
from jax.experimental import pallas as pl


def kernel(x, tables, W1, b1, W2, b2, W3, b3, W4, b4):
    raise NotImplementedError("write your pallas kernel here")


# speedup vs baseline: 3.4654x; 3.4654x over previous
import functools

import numpy as np
import jax
import jax.numpy as jnp
from jax import lax
from jax.experimental import pallas as pl
from jax.experimental.pallas import tpu as pltpu
from jax.experimental.pallas import tpu_sc as plsc

_L = 8
_H = 524288
_F = 4
_D = _L * _F
_P0 = 73856093
_P1 = 19349663
_CH = 1024
_BR = 2048


def _int_spacings():
    return [int(256 // np.power(1.6, _L - i - 1)) for i in range(_L)]


def _sc_gather_fn(B):
    info = plsc.get_sparse_core_info()
    nc = info.num_cores
    nw = nc * info.num_subcores
    bpw = B // nw
    nch = bpw // _CH
    sp = _int_spacings()
    mesh = plsc.VectorSubcoreMesh(core_axis_name="c", subcore_axis_name="s")

    @functools.partial(
        pl.kernel,
        mesh=mesh,
        compiler_params=pltpu.CompilerParams(
            use_tc_tiling_on_sc=False, needs_layout_passes=False,
        ),
        out_type=jax.ShapeDtypeStruct((B * _L, _F), jnp.float32),
        scratch_types=(
            [pltpu.VMEM((_CH,), jnp.float32) for _ in range(2)]
            + [
                pltpu.VMEM((_L * _CH,), jnp.int32),
                pltpu.VMEM((_L * _CH, _F), jnp.float32),
                pltpu.SemaphoreType.DMA,
            ]
        ),
    )
    def sc(x0_hbm, x1_hbm, tab_hbm, out_hbm, x0_v, x1_v, idx_v, hf_v, sem):
        wid = lax.axis_index("s") * jnp.int32(nc) + lax.axis_index("c")
        base = wid * jnp.int32(bpw)
        e8 = jax.lax.iota(jnp.int32, 16) * jnp.int32(_L)

        def chunk(ci, carry):
            off = base + ci * jnp.int32(_CH)
            pltpu.sync_copy(x0_hbm.at[pl.ds(off, _CH)], x0_v)
            pltpu.sync_copy(x1_hbm.at[pl.ds(off, _CH)], x1_v)

            def vec(j, c2):
                s16 = pl.ds(j * 16, 16)
                xs0 = x0_v[s16] * 0.5 + 0.5
                xs1 = x1_v[s16] * 0.5 + 0.5
                m0 = (xs0 * 16777216.0).astype(jnp.int32)
                m1 = (xs1 * 16777216.0).astype(jnp.int32)
                sh = jnp.int32(12)
                msk = jnp.int32(4095)
                mh0, ml0 = m0 >> sh, m0 & msk
                mh1, ml1 = m1 >> sh, m1 & msk
                p0 = jnp.int32(_P0)
                p1 = jnp.int32(_P1)
                hm = jnp.int32(_H - 1)
                jbase = j * jnp.int32(16 * _L)
                for l in range(_L):
                    s = jnp.int32(sp[l])
                    c0 = (mh0 * s + ((ml0 * s) >> sh)) >> sh
                    c1 = (mh1 * s + ((ml1 * s) >> sh)) >> sh
                    h = ((c0 * p0 + c1 * p1) & hm) + jnp.int32(l * _H)
                    pos = e8 + (jbase + jnp.int32(l))
                    plsc.store_scatter(idx_v, [pos], h)
                return c2

            lax.fori_loop(jnp.int32(0), jnp.int32(_CH // 16), vec,
                          jnp.int32(0))
            pltpu.async_copy(tab_hbm.at[idx_v], hf_v, sem).wait()
            pltpu.sync_copy(hf_v, out_hbm.at[pl.ds(off * _L, _CH * _L)])
            return carry

        lax.fori_loop(jnp.int32(0), jnp.int32(nch), chunk, jnp.int32(0))

    return sc


def _leaky(v):
    return jnp.where(v >= 0, v, jnp.float32(0.01) * v)


def _mlp_body(hf_ref, w1, b1, w2, b2, w3, b3, w4, b4, o_ref):
    def mm(a, w):
        return lax.dot_general(
            a, w[...], (((1,), (0,)), ((), ())),
            preferred_element_type=jnp.float32,
            precision=lax.Precision.HIGHEST,
        )

    h = _leaky(mm(hf_ref[...], w1) + b1[...])
    h = _leaky(mm(h, w2) + b2[...])
    h = _leaky(mm(h, w3) + b3[...])
    o_ref[...] = _leaky(_leaky(mm(h, w4) + b4[...]))


def kernel(x, tables, W1, b1, W2, b2, W3, b3, W4, b4):
    B = x.shape[0]
    x0 = x[:, 0]
    x1 = x[:, 1]
    tab = tables.reshape(_L * _H, _F)
    f32 = jnp.float32
    eye4 = jnp.eye(4, dtype=f32)
    W1b = jnp.kron(eye4, W1.astype(f32).T)
    W2b = jnp.kron(eye4, W2.astype(f32).T)
    W3b = jnp.kron(eye4, W3.astype(f32).T)
    W4blk = jnp.concatenate(
        [W4.astype(f32).T, jnp.zeros((8, 31), f32)], axis=1)
    W4b = jnp.kron(eye4, W4blk)
    b4blk = jnp.concatenate([b4.astype(f32), jnp.zeros((31,), f32)])
    b1b = jnp.tile(b1.astype(f32), 4).reshape(1, 128)
    b2b = jnp.tile(b2.astype(f32), 4).reshape(1, 64)
    b3b = jnp.tile(b3.astype(f32), 4).reshape(1, 32)
    b4b = jnp.tile(b4blk, 4).reshape(1, 128)

    hf = _sc_gather_fn(B)(x0, x1, tab)
    rows = B // 4
    hf = hf.reshape(rows, _D * 4) * jnp.float32(1.0)
    z = np.int32(0)
    out = pl.pallas_call(
        _mlp_body,
        grid=(rows // _BR,),
        in_specs=[
            pl.BlockSpec((_BR, 128), lambda i: (i, z)),
            pl.BlockSpec((128, 128), lambda i: (z, z)),
            pl.BlockSpec((1, 128), lambda i: (z, z)),
            pl.BlockSpec((128, 64), lambda i: (z, z)),
            pl.BlockSpec((1, 64), lambda i: (z, z)),
            pl.BlockSpec((64, 32), lambda i: (z, z)),
            pl.BlockSpec((1, 32), lambda i: (z, z)),
            pl.BlockSpec((32, 128), lambda i: (z, z)),
            pl.BlockSpec((1, 128), lambda i: (z, z)),
        ],
        out_specs=pl.BlockSpec((_BR, 128), lambda i: (i, z)),
        out_shape=jax.ShapeDtypeStruct((rows, 128), jnp.float32),
    )(hf, W1b, b1b, W2b, b2b, W3b, b3b, W4b, b4b)
    out = out.reshape(B, 32)[:, :1]
    return out.astype(jnp.float64)


# speedup vs baseline: 8.6039x; 2.4828x over previous
import functools

import numpy as np
import jax
import jax.numpy as jnp
from jax import lax
from jax.experimental import pallas as pl
from jax.experimental.pallas import tpu as pltpu
from jax.experimental.pallas import tpu_sc as plsc

_L = 8
_H = 524288
_F = 4
_D = _L * _F
_P0 = 73856093
_P1 = 19349663
_CH = 1024
_BR = 2048


def _int_spacings():
    return [int(256 // np.power(1.6, _L - i - 1)) for i in range(_L)]


def _sc_gather_fn(B):
    info = plsc.get_sparse_core_info()
    nc = info.num_cores
    nw = nc * info.num_subcores
    bpw = B // nw
    nch = bpw // _CH
    sp = _int_spacings()
    mesh = plsc.VectorSubcoreMesh(core_axis_name="c", subcore_axis_name="s")

    @functools.partial(
        pl.kernel,
        mesh=mesh,
        compiler_params=pltpu.CompilerParams(
            use_tc_tiling_on_sc=False, needs_layout_passes=False,
        ),
        out_type=jax.ShapeDtypeStruct((B * _D,), jnp.float32),
        scratch_types=(
            [pltpu.VMEM((_CH,), jnp.float32) for _ in range(2)]
            + [
                pltpu.VMEM((_D * _CH,), jnp.int32),
                pltpu.VMEM((_D * _CH,), jnp.float32),
                pltpu.SemaphoreType.DMA,
            ]
        ),
    )
    def sc(x0_hbm, x1_hbm, tab_hbm, out_hbm, x0_v, x1_v, idx_v, hf_v, sem):
        wid = lax.axis_index("s") * jnp.int32(nc) + lax.axis_index("c")
        base = wid * jnp.int32(bpw)
        e32 = jax.lax.iota(jnp.int32, 16) * jnp.int32(_D)

        def chunk(ci, carry):
            off = base + ci * jnp.int32(_CH)
            pltpu.sync_copy(x0_hbm.at[pl.ds(off, _CH)], x0_v)
            pltpu.sync_copy(x1_hbm.at[pl.ds(off, _CH)], x1_v)

            def vec(j, c2):
                s16 = pl.ds(j * 16, 16)
                xs0 = x0_v[s16] * 0.5 + 0.5
                xs1 = x1_v[s16] * 0.5 + 0.5
                m0 = (xs0 * 16777216.0).astype(jnp.int32)
                m1 = (xs1 * 16777216.0).astype(jnp.int32)
                sh = jnp.int32(12)
                msk = jnp.int32(4095)
                mh0, ml0 = m0 >> sh, m0 & msk
                mh1, ml1 = m1 >> sh, m1 & msk
                p0 = jnp.int32(_P0)
                p1 = jnp.int32(_P1)
                hm = jnp.int32(_H - 1)
                hh = jnp.int32(_H)
                jbase = j * jnp.int32(16 * _D)
                for l in range(_L):
                    s = jnp.int32(sp[l])
                    c0 = (mh0 * s + ((ml0 * s) >> sh)) >> sh
                    c1 = (mh1 * s + ((ml1 * s) >> sh)) >> sh
                    h = ((c0 * p0 + c1 * p1) & hm) + jnp.int32(l * 4 * _H)
                    for f in range(_F):
                        pos = e32 + (jbase + jnp.int32(l * _F + f))
                        plsc.store_scatter(idx_v, [pos], h)
                        if f < _F - 1:
                            h = h + hh
                return c2

            lax.fori_loop(jnp.int32(0), jnp.int32(_CH // 16), vec,
                          jnp.int32(0))
            pltpu.async_copy(tab_hbm.at[idx_v], hf_v, sem).wait()
            pltpu.sync_copy(hf_v, out_hbm.at[pl.ds(off * _D, _CH * _D)])
            return carry

        lax.fori_loop(jnp.int32(0), jnp.int32(nch), chunk, jnp.int32(0))

    return sc


def _leaky(v):
    return jnp.where(v >= 0, v, jnp.float32(0.01) * v)


def _mlp_body(hf_ref, w1, b1, w2, b2, w3, b3, w4, b4, o_ref):
    def mm(a, w):
        return lax.dot_general(
            a, w[...], (((1,), (0,)), ((), ())),
            preferred_element_type=jnp.float32,
            precision=lax.Precision.HIGHEST,
        )

    h = _leaky(mm(hf_ref[...], w1) + b1[...])
    h = _leaky(mm(h, w2) + b2[...])
    h = _leaky(mm(h, w3) + b3[...])
    o_ref[...] = _leaky(_leaky(mm(h, w4) + b4[...]))


def kernel(x, tables, W1, b1, W2, b2, W3, b3, W4, b4):
    B = x.shape[0]
    x0 = x[:, 0]
    x1 = x[:, 1]
    tab = tables.transpose(0, 2, 1).reshape(_L * _F * _H)
    f32 = jnp.float32
    eye4 = jnp.eye(4, dtype=f32)
    W1b = jnp.kron(eye4, W1.astype(f32).T)
    W2b = jnp.kron(eye4, W2.astype(f32).T)
    W3b = jnp.kron(eye4, W3.astype(f32).T)
    W4blk = jnp.concatenate(
        [W4.astype(f32).T, jnp.zeros((8, 31), f32)], axis=1)
    W4b = jnp.kron(eye4, W4blk)
    b4blk = jnp.concatenate([b4.astype(f32), jnp.zeros((31,), f32)])
    b1b = jnp.tile(b1.astype(f32), 4).reshape(1, 128)
    b2b = jnp.tile(b2.astype(f32), 4).reshape(1, 64)
    b3b = jnp.tile(b3.astype(f32), 4).reshape(1, 32)
    b4b = jnp.tile(b4blk, 4).reshape(1, 128)

    hf = _sc_gather_fn(B)(x0, x1, tab)
    rows = B // 4
    hf = hf.reshape(rows, _D * 4)
    z = np.int32(0)
    out = pl.pallas_call(
        _mlp_body,
        grid=(rows // _BR,),
        in_specs=[
            pl.BlockSpec((_BR, 128), lambda i: (i, z)),
            pl.BlockSpec((128, 128), lambda i: (z, z)),
            pl.BlockSpec((1, 128), lambda i: (z, z)),
            pl.BlockSpec((128, 64), lambda i: (z, z)),
            pl.BlockSpec((1, 64), lambda i: (z, z)),
            pl.BlockSpec((64, 32), lambda i: (z, z)),
            pl.BlockSpec((1, 32), lambda i: (z, z)),
            pl.BlockSpec((32, 128), lambda i: (z, z)),
            pl.BlockSpec((1, 128), lambda i: (z, z)),
        ],
        out_specs=pl.BlockSpec((_BR, 128), lambda i: (i, z)),
        out_shape=jax.ShapeDtypeStruct((rows, 128), jnp.float32),
    )(hf, W1b, b1b, W2b, b2b, W3b, b3b, W4b, b4b)
    out = out.reshape(B, 32)[:, :1]
    return out.astype(jnp.float64)


# speedup vs baseline: 33.0912x; 3.8461x over previous
import functools

import numpy as np
import jax
import jax.numpy as jnp
from jax import lax
from jax.experimental import pallas as pl
from jax.experimental.pallas import tpu as pltpu
from jax.experimental.pallas import tpu_sc as plsc

_L = 8
_H = 524288
_F = 4
_D = _L * _F
_P0 = 73856093
_P1 = 19349663
_CH = 1024
_BR = 2048


def _int_spacings():
    return [int(256 // np.power(1.6, _L - i - 1)) for i in range(_L)]


def _sc_gather_fn(B):
    info = plsc.get_sparse_core_info()
    nc = info.num_cores
    nw = nc * info.num_subcores
    bpw = B // nw
    nch = bpw // _CH
    sp = _int_spacings()
    mesh = plsc.VectorSubcoreMesh(core_axis_name="c", subcore_axis_name="s")

    @functools.partial(
        pl.kernel,
        mesh=mesh,
        compiler_params=pltpu.CompilerParams(
            use_tc_tiling_on_sc=False, needs_layout_passes=False,
        ),
        out_type=jax.ShapeDtypeStruct((B * _D,), jnp.float32),
        scratch_types=(
            [pltpu.VMEM((_CH,), jnp.float32) for _ in range(2)]
            + [
                pltpu.VMEM((_D * _CH,), jnp.int32),
                pltpu.VMEM((_D * _CH,), jnp.float32),
                pltpu.SemaphoreType.DMA,
            ]
        ),
    )
    def sc(x0_hbm, x1_hbm, tab_hbm, out_hbm, x0_v, x1_v, idx_v, hf_v, sem):
        wid = lax.axis_index("s") * jnp.int32(nc) + lax.axis_index("c")
        base = wid * jnp.int32(bpw)
        e32 = jax.lax.iota(jnp.int32, 16) * jnp.int32(_D)

        def chunk(ci, carry):
            off = base + ci * jnp.int32(_CH)
            pltpu.sync_copy(x0_hbm.at[pl.ds(off, _CH)], x0_v)
            pltpu.sync_copy(x1_hbm.at[pl.ds(off, _CH)], x1_v)

            def vec(j, c2):
                s16 = pl.ds(j * 16, 16)
                xs0 = x0_v[s16] * 0.5 + 0.5
                xs1 = x1_v[s16] * 0.5 + 0.5
                m0 = (xs0 * 16777216.0).astype(jnp.int32)
                m1 = (xs1 * 16777216.0).astype(jnp.int32)
                sh = jnp.int32(12)
                msk = jnp.int32(4095)
                mh0, ml0 = m0 >> sh, m0 & msk
                mh1, ml1 = m1 >> sh, m1 & msk
                p0 = jnp.int32(_P0)
                p1 = jnp.int32(_P1)
                hm = jnp.int32(_H - 1)
                hh = jnp.int32(_H)
                jbase = j * jnp.int32(16 * _D)
                for l in range(_L):
                    s = jnp.int32(sp[l])
                    c0 = (mh0 * s + ((ml0 * s) >> sh)) >> sh
                    c1 = (mh1 * s + ((ml1 * s) >> sh)) >> sh
                    h = ((c0 * p0 + c1 * p1) & hm) + jnp.int32(l * 4 * _H)
                    for f in range(_F):
                        pos = e32 + (jbase + jnp.int32(l * _F + f))
                        plsc.store_scatter(idx_v, [pos], h)
                        if f < _F - 1:
                            h = h + hh
                return c2

            lax.fori_loop(jnp.int32(0), jnp.int32(_CH // 16), vec,
                          jnp.int32(0))
            pltpu.async_copy(tab_hbm.at[idx_v], hf_v, sem).wait()
            pltpu.sync_copy(hf_v, out_hbm.at[pl.ds(off * _D, _CH * _D)])
            return carry

        lax.fori_loop(jnp.int32(0), jnp.int32(nch), chunk, jnp.int32(0))

    return sc


def _leaky(v):
    return jnp.where(v >= 0, v, jnp.float32(0.01) * v)


def _mlp_body(hf_ref, w1, b1, w2, b2, w3, b3, w4, b4, o_ref):
    def mm(a, w):
        return lax.dot_general(
            a, w[...], (((1,), (0,)), ((), ())),
            preferred_element_type=jnp.float32,
            precision=lax.Precision.HIGHEST,
        )

    h = _leaky(mm(hf_ref[...], w1) + b1[...])
    h = _leaky(mm(h, w2) + b2[...])
    h = _leaky(mm(h, w3) + b3[...])
    o_ref[...] = _leaky(_leaky(mm(h, w4) + b4[...]))


def kernel(x, tables, W1, b1, W2, b2, W3, b3, W4, b4):
    B = x.shape[0]
    x0 = x[:, 0]
    x1 = x[:, 1]
    tab = tables.transpose(0, 2, 1).reshape(_L * _F * _H)
    f32 = jnp.float32
    eye4 = jnp.eye(4, dtype=f32)
    W1b = jnp.kron(eye4, W1.astype(f32).T)
    W2b = jnp.kron(eye4, W2.astype(f32).T)
    W3b = jnp.kron(eye4, W3.astype(f32).T)
    W4b = jnp.zeros((32, 128), f32)
    w4row = W4.astype(f32).reshape(8)
    for p in range(4):
        W4b = W4b.at[8 * p:8 * p + 8, p].set(w4row)
    b4b = jnp.zeros((1, 128), f32).at[0, :4].set(b4.astype(f32)[0])
    b1b = jnp.tile(b1.astype(f32), 4).reshape(1, 128)
    b2b = jnp.tile(b2.astype(f32), 4).reshape(1, 64)
    b3b = jnp.tile(b3.astype(f32), 4).reshape(1, 32)

    hf = _sc_gather_fn(B)(x0, x1, tab)
    rows = B // 4
    hf = hf.reshape(rows, _D * 4)
    z = np.int32(0)
    out = pl.pallas_call(
        _mlp_body,
        grid=(rows // _BR,),
        in_specs=[
            pl.BlockSpec((_BR, 128), lambda i: (i, z)),
            pl.BlockSpec((128, 128), lambda i: (z, z)),
            pl.BlockSpec((1, 128), lambda i: (z, z)),
            pl.BlockSpec((128, 64), lambda i: (z, z)),
            pl.BlockSpec((1, 64), lambda i: (z, z)),
            pl.BlockSpec((64, 32), lambda i: (z, z)),
            pl.BlockSpec((1, 32), lambda i: (z, z)),
            pl.BlockSpec((32, 128), lambda i: (z, z)),
            pl.BlockSpec((1, 128), lambda i: (z, z)),
        ],
        out_specs=pl.BlockSpec((_BR, 128), lambda i: (i, z)),
        out_shape=jax.ShapeDtypeStruct((rows, 128), jnp.float32),
    )(hf, W1b, b1b, W2b, b2b, W3b, b3b, W4b, b4b)
    out = out[:, :4].reshape(B, 1)
    return out.astype(jnp.float64)


# speedup vs baseline: 37.7396x; 1.1405x over previous
import functools

import numpy as np
import jax
import jax.numpy as jnp
from jax import lax
from jax.experimental import pallas as pl
from jax.experimental.pallas import tpu as pltpu
from jax.experimental.pallas import tpu_sc as plsc

_L = 8
_H = 524288
_F = 4
_D = _L * _F
_P0 = 73856093
_P1 = 19349663
_CH = 1024
_BR = 2048


def _int_spacings():
    return [int(256 // np.power(1.6, _L - i - 1)) for i in range(_L)]


def _sc_gather_fn(B):
    info = plsc.get_sparse_core_info()
    nc = info.num_cores
    nw = nc * info.num_subcores
    bpw = B // nw
    nch = bpw // _CH
    sp = _int_spacings()
    mesh = plsc.VectorSubcoreMesh(core_axis_name="c", subcore_axis_name="s")

    @functools.partial(
        pl.kernel,
        mesh=mesh,
        compiler_params=pltpu.CompilerParams(
            use_tc_tiling_on_sc=False, needs_layout_passes=False,
        ),
        out_type=jax.ShapeDtypeStruct((B * _D,), jnp.float32),
        scratch_types=(
            [pltpu.VMEM((_CH,), jnp.float32) for _ in range(2)]
            + [
                pltpu.VMEM((_D * _CH,), jnp.int32),
                pltpu.VMEM((_D * _CH,), jnp.float32),
                pltpu.SemaphoreType.DMA,
            ]
        ),
    )
    def sc(x0_hbm, x1_hbm, tab_hbm, out_hbm, x0_v, x1_v, idx_v, hf_v, sem):
        wid = lax.axis_index("s") * jnp.int32(nc) + lax.axis_index("c")
        base = wid * jnp.int32(bpw)
        e32 = jax.lax.iota(jnp.int32, 16) * jnp.int32(_D)

        def chunk(ci, carry):
            off = base + ci * jnp.int32(_CH)
            pltpu.sync_copy(x0_hbm.at[pl.ds(off, _CH)], x0_v)
            pltpu.sync_copy(x1_hbm.at[pl.ds(off, _CH)], x1_v)

            def vec(j, c2):
                s16 = pl.ds(j * 16, 16)
                xs0 = x0_v[s16] * 0.5 + 0.5
                xs1 = x1_v[s16] * 0.5 + 0.5
                m0 = (xs0 * 16777216.0).astype(jnp.int32)
                m1 = (xs1 * 16777216.0).astype(jnp.int32)
                sh = jnp.int32(12)
                msk = jnp.int32(4095)
                mh0, ml0 = m0 >> sh, m0 & msk
                mh1, ml1 = m1 >> sh, m1 & msk
                p0 = jnp.int32(_P0)
                p1 = jnp.int32(_P1)
                hm = jnp.int32(_H - 1)
                hh = jnp.int32(_H)
                jbase = j * jnp.int32(16 * _D)
                for l in range(_L):
                    s = jnp.int32(sp[l])
                    c0 = (mh0 * s + ((ml0 * s) >> sh)) >> sh
                    c1 = (mh1 * s + ((ml1 * s) >> sh)) >> sh
                    h = ((c0 * p0 + c1 * p1) & hm) + jnp.int32(l * 4 * _H)
                    for f in range(_F):
                        pos = e32 + (jbase + jnp.int32(l * _F + f))
                        plsc.store_scatter(idx_v, [pos], h)
                        if f < _F - 1:
                            h = h + hh
                return c2

            lax.fori_loop(jnp.int32(0), jnp.int32(_CH // 16), vec,
                          jnp.int32(0))
            pltpu.async_copy(tab_hbm.at[idx_v], hf_v, sem).wait()
            pltpu.sync_copy(hf_v, out_hbm.at[pl.ds(off * _D, _CH * _D)])
            return carry

        lax.fori_loop(jnp.int32(0), jnp.int32(nch), chunk, jnp.int32(0))

    return sc


def _leaky(v):
    return jnp.where(v >= 0, v, jnp.float32(0.01) * v)


def _mlp_body(hf_ref, w1, b1, w2, b2, w3, b3, w4, b4, o_ref):
    def mm(a, w):
        return lax.dot_general(
            a, w[...], (((1,), (0,)), ((), ())),
            preferred_element_type=jnp.float32,
            precision=lax.Precision.HIGHEST,
        )

    h = _leaky(mm(hf_ref[...], w1) + b1[...])
    h = _leaky(mm(h, w2) + b2[...])
    h = _leaky(mm(h, w3) + b3[...])
    o_ref[...] = _leaky(_leaky(mm(h, w4) + b4[...]))


def kernel(x, tables, W1, b1, W2, b2, W3, b3, W4, b4):
    B = x.shape[0]
    x0 = x[:, 0]
    x1 = x[:, 1]
    tab = tables.transpose(0, 2, 1).reshape(_L * _F * _H)
    f32 = jnp.float32
    eye4 = jnp.eye(4, dtype=f32)
    W1b = jnp.kron(eye4, W1.astype(f32).T)
    W2b = jnp.kron(eye4, W2.astype(f32).T)
    W3b = jnp.kron(eye4, W3.astype(f32).T)
    W4b = jnp.zeros((32, 128), f32)
    w4row = W4.astype(f32).reshape(8)
    for p in range(4):
        W4b = W4b.at[8 * p:8 * p + 8, p].set(w4row)
    b4b = jnp.zeros((1, 128), f32).at[0, :4].set(b4.astype(f32)[0])
    b1b = jnp.tile(b1.astype(f32), 4).reshape(1, 128)
    b2b = jnp.tile(b2.astype(f32), 4).reshape(1, 64)
    b3b = jnp.tile(b3.astype(f32), 4).reshape(1, 32)

    nsl = 2
    Bs = B // nsl
    sc_fn = _sc_gather_fn(Bs)
    z = np.int32(0)
    outs = []
    for k in range(nsl):
        sl = slice(k * Bs, (k + 1) * Bs)
        hf = sc_fn(x0[sl], x1[sl], tab)
        rows = Bs // 4
        hf = hf.reshape(rows, _D * 4)
        out = pl.pallas_call(
            _mlp_body,
            grid=(rows // _BR,),
            in_specs=[
                pl.BlockSpec((_BR, 128), lambda i: (i, z)),
                pl.BlockSpec((128, 128), lambda i: (z, z)),
                pl.BlockSpec((1, 128), lambda i: (z, z)),
                pl.BlockSpec((128, 64), lambda i: (z, z)),
                pl.BlockSpec((1, 64), lambda i: (z, z)),
                pl.BlockSpec((64, 32), lambda i: (z, z)),
                pl.BlockSpec((1, 32), lambda i: (z, z)),
                pl.BlockSpec((32, 128), lambda i: (z, z)),
                pl.BlockSpec((1, 128), lambda i: (z, z)),
            ],
            out_specs=pl.BlockSpec((_BR, 128), lambda i: (i, z)),
            out_shape=jax.ShapeDtypeStruct((rows, 128), jnp.float32),
        )(hf, W1b, b1b, W2b, b2b, W3b, b3b, W4b, b4b)
        outs.append(out[:, :4].reshape(Bs, 1))
    out = jnp.concatenate(outs, axis=0)
    return out.astype(jnp.float64)


# speedup vs baseline: 40.3768x; 1.0699x over previous
import functools

import numpy as np
import jax
import jax.numpy as jnp
from jax import lax
from jax.experimental import pallas as pl
from jax.experimental.pallas import tpu as pltpu
from jax.experimental.pallas import tpu_sc as plsc

_L = 8
_H = 524288
_F = 4
_D = _L * _F
_P0 = 73856093
_P1 = 19349663
_CH = 1024
_BR = 2048


def _int_spacings():
    return [int(256 // np.power(1.6, _L - i - 1)) for i in range(_L)]


def _sc_gather_fn(B):
    info = plsc.get_sparse_core_info()
    nc = info.num_cores
    nw = nc * info.num_subcores
    bpw = B // nw
    nch = bpw // _CH
    sp = _int_spacings()
    mesh = plsc.VectorSubcoreMesh(core_axis_name="c", subcore_axis_name="s")

    @functools.partial(
        pl.kernel,
        mesh=mesh,
        compiler_params=pltpu.CompilerParams(
            use_tc_tiling_on_sc=False, needs_layout_passes=False,
        ),
        out_type=jax.ShapeDtypeStruct((B * _D,), jnp.float32),
        scratch_types=(
            [pltpu.VMEM((_CH,), jnp.float32) for _ in range(2)]
            + [
                pltpu.VMEM((_D * _CH,), jnp.int32),
                pltpu.VMEM((_D * _CH,), jnp.float32),
                pltpu.SemaphoreType.DMA,
            ]
        ),
    )
    def sc(x0_hbm, x1_hbm, tab_hbm, out_hbm, x0_v, x1_v, idx_v, hf_v, sem):
        wid = lax.axis_index("s") * jnp.int32(nc) + lax.axis_index("c")
        base = wid * jnp.int32(bpw)
        e32 = jax.lax.iota(jnp.int32, 16) * jnp.int32(_D)

        def chunk(ci, carry):
            off = base + ci * jnp.int32(_CH)
            pltpu.sync_copy(x0_hbm.at[pl.ds(off, _CH)], x0_v)
            pltpu.sync_copy(x1_hbm.at[pl.ds(off, _CH)], x1_v)

            def vec(j, c2):
                s16 = pl.ds(j * 16, 16)
                xs0 = x0_v[s16] * 0.5 + 0.5
                xs1 = x1_v[s16] * 0.5 + 0.5
                m0 = (xs0 * 16777216.0).astype(jnp.int32)
                m1 = (xs1 * 16777216.0).astype(jnp.int32)
                sh = jnp.int32(12)
                msk = jnp.int32(4095)
                mh0, ml0 = m0 >> sh, m0 & msk
                mh1, ml1 = m1 >> sh, m1 & msk
                p0 = jnp.int32(_P0)
                p1 = jnp.int32(_P1)
                hm = jnp.int32(_H - 1)
                hh = jnp.int32(_H)
                jbase = j * jnp.int32(16 * _D)
                for l in range(_L):
                    s = jnp.int32(sp[l])
                    c0 = (mh0 * s + ((ml0 * s) >> sh)) >> sh
                    c1 = (mh1 * s + ((ml1 * s) >> sh)) >> sh
                    h = ((c0 * p0 + c1 * p1) & hm) + jnp.int32(l * 4 * _H)
                    for f in range(_F):
                        pos = e32 + (jbase + jnp.int32(l * _F + f))
                        plsc.store_scatter(idx_v, [pos], h)
                        if f < _F - 1:
                            h = h + hh
                return c2

            lax.fori_loop(jnp.int32(0), jnp.int32(_CH // 16), vec,
                          jnp.int32(0))
            pltpu.async_copy(tab_hbm.at[idx_v], hf_v, sem).wait()
            pltpu.sync_copy(hf_v, out_hbm.at[pl.ds(off * _D, _CH * _D)])
            return carry

        lax.fori_loop(jnp.int32(0), jnp.int32(nch), chunk, jnp.int32(0))

    return sc


def _leaky(v):
    return jnp.where(v >= 0, v, jnp.float32(0.01) * v)


def _mlp_body(hf_ref, w1, b1, w2, b2, w3, b3, w4, b4, o_ref):
    def mm(a, w):
        return lax.dot_general(
            a, w[...], (((1,), (0,)), ((), ())),
            preferred_element_type=jnp.float32,
            precision=lax.Precision.HIGHEST,
        )

    h = _leaky(mm(hf_ref[...], w1) + b1[...])
    h = _leaky(mm(h, w2) + b2[...])
    h = _leaky(mm(h, w3) + b3[...])
    o_ref[...] = _leaky(_leaky(mm(h, w4) + b4[...]))


def kernel(x, tables, W1, b1, W2, b2, W3, b3, W4, b4):
    B = x.shape[0]
    x0 = x[:, 0]
    x1 = x[:, 1]
    tab = tables.transpose(0, 2, 1).reshape(_L * _F * _H)
    f32 = jnp.float32
    eye4 = jnp.eye(4, dtype=f32)
    W1b = jnp.kron(eye4, W1.astype(f32).T)
    W2b = jnp.kron(eye4, W2.astype(f32).T)
    W3b = jnp.kron(eye4, W3.astype(f32).T)
    W4b = jnp.zeros((32, 128), f32)
    w4row = W4.astype(f32).reshape(8)
    for p in range(4):
        W4b = W4b.at[8 * p:8 * p + 8, p].set(w4row)
    b4b = jnp.zeros((1, 128), f32).at[0, :4].set(b4.astype(f32)[0])
    b1b = jnp.tile(b1.astype(f32), 4).reshape(1, 128)
    b2b = jnp.tile(b2.astype(f32), 4).reshape(1, 64)
    b3b = jnp.tile(b3.astype(f32), 4).reshape(1, 32)

    nsl = 4
    Bs = B // nsl
    sc_fn = _sc_gather_fn(Bs)
    z = np.int32(0)
    outs = []
    for k in range(nsl):
        sl = slice(k * Bs, (k + 1) * Bs)
        hf = sc_fn(x0[sl], x1[sl], tab)
        rows = Bs // 4
        hf = hf.reshape(rows, _D * 4)
        out = pl.pallas_call(
            _mlp_body,
            grid=(rows // _BR,),
            in_specs=[
                pl.BlockSpec((_BR, 128), lambda i: (i, z)),
                pl.BlockSpec((128, 128), lambda i: (z, z)),
                pl.BlockSpec((1, 128), lambda i: (z, z)),
                pl.BlockSpec((128, 64), lambda i: (z, z)),
                pl.BlockSpec((1, 64), lambda i: (z, z)),
                pl.BlockSpec((64, 32), lambda i: (z, z)),
                pl.BlockSpec((1, 32), lambda i: (z, z)),
                pl.BlockSpec((32, 128), lambda i: (z, z)),
                pl.BlockSpec((1, 128), lambda i: (z, z)),
            ],
            out_specs=pl.BlockSpec((_BR, 128), lambda i: (i, z)),
            out_shape=jax.ShapeDtypeStruct((rows, 128), jnp.float32),
        )(hf, W1b, b1b, W2b, b2b, W3b, b3b, W4b, b4b)
        outs.append(out[:, :4].reshape(Bs, 1))
    out = jnp.concatenate(outs, axis=0)
    return out.astype(jnp.float64)
